# baseline (speedup 1.0000x reference)
import jax
import jax.numpy as jnp
from jax import lax
from jax.experimental import pallas as pl
from jax.experimental.pallas import tpu as pltpu

B = 64
N = 1_000_000
TOPK = 5
BN = 49152
GRID = (N + BN - 1) // BN
LAST_VALID = N - (GRID - 1) * BN


def _body(lab_sref, hbm_ref, lab_ref, logits_ref, out_ref,
          acc_ref, x_ref, stage_ref, sem):
    j = pl.program_id(0)

    @pl.when(j == 0)
    def _():
        copies = []
        for k in range(B):
            lane0 = (lab_sref[k] // 128) * 128
            copies.append(pltpu.make_async_copy(
                hbm_ref.at[pl.ds((k // 8) * 8, 8), pl.ds(lane0, 128)],
                stage_ref.at[k],
                sem))
        for c in copies:
            c.start()
        for c in copies:
            c.wait()
        rows = [stage_ref[k, (k % 8):(k % 8) + 1, :] for k in range(B)]
        v = jnp.concatenate(rows, axis=0)
        off = lab_ref[...] % 128
        lanes = lax.broadcasted_iota(jnp.int32, (B, 128), 1)
        x_ref[...] = jnp.sum(jnp.where(lanes == off, v, 0.0), axis=1)[:, None]
        acc_ref[...] = jnp.zeros_like(acc_ref)

    v = logits_ref[...]
    x = x_ref[...]
    lab = lab_ref[...]
    cols = j * BN + lax.broadcasted_iota(jnp.int32, (B, BN), 1)
    beats = (v > x) | ((v == x) & (cols < lab))

    @pl.when(j < GRID - 1)
    def _():
        acc_ref[...] += jnp.sum(beats.astype(jnp.int32), axis=1)[:, None]

    @pl.when(j == GRID - 1)
    def _():
        valid = cols < N
        acc = acc_ref[...] + jnp.sum((beats & valid).astype(jnp.int32), axis=1)[:, None]
        out_ref[0, 0] = jnp.sum((acc < TOPK).astype(jnp.float32)) * (1.0 / B)


_fused = pl.pallas_call(
    _body,
    grid_spec=pltpu.PrefetchScalarGridSpec(
        num_scalar_prefetch=1,
        grid=(GRID,),
        in_specs=[
            pl.BlockSpec(memory_space=pltpu.MemorySpace.HBM),
            pl.BlockSpec((B, 1), lambda j, lab: (0, 0)),
            pl.BlockSpec((B, BN), lambda j, lab: (0, j)),
        ],
        out_specs=pl.BlockSpec(memory_space=pltpu.SMEM),
        scratch_shapes=[
            pltpu.VMEM((B, 1), jnp.int32),
            pltpu.VMEM((B, 1), jnp.float32),
            pltpu.VMEM((B, 8, 128), jnp.float32),
            pltpu.SemaphoreType.DMA,
        ],
    ),
    out_shape=jax.ShapeDtypeStruct((1, 1), jnp.float32),
)


def kernel(logits, labels):
    out = _fused(labels, logits, labels[:, None], logits)
    return out[0, 0]


# speedup vs baseline: 1.0047x; 1.0047x over previous
import jax
import jax.numpy as jnp
from jax import lax
from jax.experimental import pallas as pl
from jax.experimental.pallas import tpu as pltpu

B = 64
N = 1_000_000
TOPK = 5
BN = 40960
GRID = (N + BN - 1) // BN
LAST_VALID = N - (GRID - 1) * BN


def _body(lab_sref, hbm_ref, lab_ref, logits_ref, out_ref,
          acc_ref, x_ref, stage_ref, sem):
    j = pl.program_id(0)

    @pl.when(j == 0)
    def _():
        copies = []
        for k in range(B):
            lane0 = (lab_sref[k] // 128) * 128
            copies.append(pltpu.make_async_copy(
                hbm_ref.at[pl.ds((k // 8) * 8, 8), pl.ds(lane0, 128)],
                stage_ref.at[k],
                sem))
        for c in copies:
            c.start()
        for c in copies:
            c.wait()
        rows = [stage_ref[k, (k % 8):(k % 8) + 1, :] for k in range(B)]
        v = jnp.concatenate(rows, axis=0)
        off = lab_ref[...] % 128
        lanes = lax.broadcasted_iota(jnp.int32, (B, 128), 1)
        x_ref[...] = jnp.sum(jnp.where(lanes == off, v, 0.0), axis=1)[:, None]
        acc_ref[...] = jnp.zeros_like(acc_ref)

    v = logits_ref[...]
    x = x_ref[...]
    lab = lab_ref[...]
    cols = j * BN + lax.broadcasted_iota(jnp.int32, (B, BN), 1)
    beats = (v > x) | ((v == x) & (cols < lab))

    @pl.when(j < GRID - 1)
    def _():
        acc_ref[...] += jnp.sum(beats.astype(jnp.int32), axis=1)[:, None]

    @pl.when(j == GRID - 1)
    def _():
        valid = cols < N
        acc = acc_ref[...] + jnp.sum((beats & valid).astype(jnp.int32), axis=1)[:, None]
        out_ref[0, 0] = jnp.sum((acc < TOPK).astype(jnp.float32)) * (1.0 / B)


_fused = pl.pallas_call(
    _body,
    grid_spec=pltpu.PrefetchScalarGridSpec(
        num_scalar_prefetch=1,
        grid=(GRID,),
        in_specs=[
            pl.BlockSpec(memory_space=pltpu.MemorySpace.HBM),
            pl.BlockSpec((B, 1), lambda j, lab: (0, 0)),
            pl.BlockSpec((B, BN), lambda j, lab: (0, j)),
        ],
        out_specs=pl.BlockSpec(memory_space=pltpu.SMEM),
        scratch_shapes=[
            pltpu.VMEM((B, 1), jnp.int32),
            pltpu.VMEM((B, 1), jnp.float32),
            pltpu.VMEM((B, 8, 128), jnp.float32),
            pltpu.SemaphoreType.DMA,
        ],
    ),
    out_shape=jax.ShapeDtypeStruct((1, 1), jnp.float32),
)


def kernel(logits, labels):
    out = _fused(labels, logits, labels[:, None], logits)
    return out[0, 0]
